# Initial kernel scaffold; baseline (speedup 1.0000x reference)
import jax
import jax.numpy as jnp
from jax.experimental import pallas as pl


def kernel(n_feat, edge_index, W1, b1, W2, b2, W3, b3, W4, b4):
    raise NotImplementedError("write your pallas kernel here")


# speedup vs baseline: 6.6725x; 6.6725x over previous
import functools

import jax
import jax.numpy as jnp
from jax import lax
from jax.experimental import pallas as pl
from jax.experimental.pallas import tpu as pltpu
from jax.experimental.pallas import tpu_sc as plsc

NC = 2
NS = 16
NW = NC * NS


def _sc_segment_sum(n_feat, src, dst, zeros):
    N, D = n_feat.shape
    E = src.shape[0]
    assert E % NW == 0 and N % NS == 0
    epw = E // NW
    CH = 128
    full = epw // CH
    tail = epw % CH
    assert tail % 8 == 0
    rpt = (N // NS) & ~7
    rlast = N - rpt * (NS - 1)
    assert rlast % 8 == 0 and rlast > 0

    mesh = plsc.VectorSubcoreMesh(
        core_axis_name="c", subcore_axis_name="s", num_cores=NC, num_subcores=NS)

    @functools.partial(
        pl.kernel,
        out_type=jax.ShapeDtypeStruct((NC, N, D), jnp.float32),
        mesh=mesh,
        scratch_types=[
            pltpu.VMEM((CH,), jnp.int32),
            pltpu.VMEM((CH,), jnp.int32),
            pltpu.VMEM((CH, D), jnp.float32),
            pltpu.VMEM((tail,), jnp.int32) if tail else None,
            pltpu.VMEM((tail,), jnp.int32) if tail else None,
            pltpu.VMEM((tail, D), jnp.float32) if tail else None,
            pltpu.VMEM_SHARED((N, D), jnp.float32),
            pltpu.SemaphoreType.DMA,
        ],
    )
    def seg_sum(nf_hbm, src_hbm, dst_hbm, z_hbm, out_hbm,
                sidx, didx, rows, sidx_t, didx_t, rows_t, acc, sem):
        c = lax.axis_index("c")
        s = lax.axis_index("s")
        wid = s * NC + c
        r0 = pl.multiple_of(s * rpt, 8)

        @pl.when(s < NS - 1)
        def _():
            pltpu.sync_copy(z_hbm.at[pl.ds(r0, rpt)], acc.at[pl.ds(r0, rpt)])

        @pl.when(s == NS - 1)
        def _():
            pltpu.sync_copy(z_hbm.at[pl.ds(r0, rlast)], acc.at[pl.ds(r0, rlast)])

        plsc.subcore_barrier()

        base = wid * epw

        def chunk(i, _):
            off = pl.multiple_of(base + i * CH, 8)
            pltpu.sync_copy(src_hbm.at[pl.ds(off, CH)], sidx)
            pltpu.sync_copy(dst_hbm.at[pl.ds(off, CH)], didx)
            pltpu.async_copy(nf_hbm.at[sidx], rows, sem).wait()
            pltpu.sync_copy(rows, acc.at[didx], add=True)
            return 0

        lax.fori_loop(0, full, chunk, 0)

        if tail:
            off = pl.multiple_of(base + full * CH, 8)
            pltpu.sync_copy(src_hbm.at[pl.ds(off, tail)], sidx_t)
            pltpu.sync_copy(dst_hbm.at[pl.ds(off, tail)], didx_t)
            pltpu.async_copy(nf_hbm.at[sidx_t], rows_t, sem).wait()
            pltpu.sync_copy(rows_t, acc.at[didx_t], add=True)

        plsc.subcore_barrier()

        @pl.when(s < NS - 1)
        def _():
            pltpu.sync_copy(acc.at[pl.ds(r0, rpt)], out_hbm.at[c, pl.ds(r0, rpt)])

        @pl.when(s == NS - 1)
        def _():
            pltpu.sync_copy(acc.at[pl.ds(r0, rlast)],
                            out_hbm.at[c, pl.ds(r0, rlast)])

    return seg_sum(n_feat, src, dst, zeros)


def _tc_mlp(partials, W1, b1, W2, b2, W3, b3, W4, b4):
    _, N, D = partials.shape
    H = W1.shape[1]
    NB = W2.shape[1]
    BN = 1000
    assert N % BN == 0
    grid = N // BN

    def body(p_ref, W1_ref, b1_ref, W2_ref, b2_ref, W3_ref, b3_ref,
             W4_ref, b4_ref, fps_ref, out_ref, acc_ref):
        i = pl.program_id(0)
        h = p_ref[0] + p_ref[1]
        r = jnp.maximum(
            jnp.dot(h, W1_ref[...], preferred_element_type=jnp.float32)
            + b1_ref[...], 0.0)
        lg = (jnp.dot(r, W2_ref[...], preferred_element_type=jnp.float32)
              + b2_ref[...])
        m = jnp.max(lg, axis=1, keepdims=True)
        e = jnp.exp(lg - m)
        p = e / jnp.sum(e, axis=1, keepdims=True)
        colsum = jnp.sum(p, axis=0, keepdims=True)

        @pl.when(i == 0)
        def _():
            acc_ref[...] = colsum

        @pl.when(i > 0)
        def _():
            acc_ref[...] += colsum

        @pl.when(i == pl.num_programs(0) - 1)
        def _():
            fps = acc_ref[...] * 2.0
            fps_ref[...] = fps
            o = jnp.maximum(
                jnp.dot(fps, W3_ref[...], preferred_element_type=jnp.float32)
                + b3_ref[...], 0.0)
            out_ref[...] = (
                jnp.dot(o, W4_ref[...], preferred_element_type=jnp.float32)
                + b4_ref[...])

    fixed = lambda *_: (0, 0)
    return pl.pallas_call(
        body,
        grid=(grid,),
        in_specs=[
            pl.BlockSpec((2, BN, D), lambda i: (0, i, 0)),
            pl.BlockSpec((D, H), fixed),
            pl.BlockSpec((1, H), fixed),
            pl.BlockSpec((H, NB), fixed),
            pl.BlockSpec((1, NB), fixed),
            pl.BlockSpec((NB, H), fixed),
            pl.BlockSpec((1, H), fixed),
            pl.BlockSpec((H, 1), fixed),
            pl.BlockSpec((1, 1), fixed),
        ],
        out_specs=[
            pl.BlockSpec((1, NB), fixed),
            pl.BlockSpec((1, 1), fixed),
        ],
        out_shape=[
            jax.ShapeDtypeStruct((1, NB), jnp.float32),
            jax.ShapeDtypeStruct((1, 1), jnp.float32),
        ],
        scratch_shapes=[pltpu.VMEM((1, NB), jnp.float32)],
    )(partials, W1, b1.reshape(1, H), W2, b2.reshape(1, NB),
      W3, b3.reshape(1, H), W4, b4.reshape(1, 1))


def kernel(n_feat, edge_index, W1, b1, W2, b2, W3, b3, W4, b4):
    N, D = n_feat.shape
    src = edge_index[0]
    dst = edge_index[1]
    zeros = jnp.zeros((N, D), dtype=jnp.float32)
    partials = _sc_segment_sum(n_feat, src, dst, zeros)
    fps, out = _tc_mlp(partials, W1, b1, W2, b2, W3, b3, W4, b4)
    return (fps, out.squeeze(0))
